# baseline (speedup 1.0000x reference)
import functools

import jax
import jax.numpy as jnp
from jax import lax
from jax.experimental import pallas as pl
from jax.experimental.pallas import tpu as pltpu
from jax.experimental.pallas import tpu_sc as plsc

N = 10000
E = 320000
D = 128
H = 128
K = 5000
NPAD = 10240
NC = 2
NS = 16
LANES = 16
NW = NC * NS
CB = 128
EPW = 10240
NCH = EPW // CB
RPT = NPAD // NS
ZROWS = 128
INT_MIN = -(2 ** 31)


NPH = 2
NCHP = NCH // NPH


def _seg128_body(src_hbm, dst_hbm, tab_hbm, out_hbm, srcv, dstv,
                 b0, b1, acc, gsem0, gsem1, ssem0, ssem1):
    c = lax.axis_index("c")
    s = lax.axis_index("s")
    wid = c * NS + s

    zero = jnp.zeros((LANES,), jnp.float32)

    def zfill(i, _):
        r = i // (H // LANES)
        col = (i % (H // LANES)) * LANES
        b0[r, pl.ds(col, LANES)] = zero
        return 0

    lax.fori_loop(0, ZROWS * (H // LANES), zfill, 0)

    def zcp(i, _):
        pltpu.sync_copy(b0, acc.at[pl.ds(s * RPT + i * ZROWS, ZROWS), :])
        return 0

    lax.fori_loop(0, RPT // ZROWS, zcp, 0)
    plsc.subcore_barrier()

    for ph in range(NPH):
        pltpu.sync_copy(src_hbm.at[wid, pl.ds(ph * NCHP, NCHP)], srcv)
        pltpu.sync_copy(dst_hbm.at[wid, pl.ds(ph * NCHP, NCHP)], dstv)
        pltpu.async_copy(tab_hbm.at[srcv.at[0]], b0, gsem0)

        def blk(i, _):
            for t, (bw, gsem, ssem, bo, osem) in enumerate(
                    ((b0, gsem0, ssem0, b1, ssem1),
                     (b1, gsem1, ssem1, b0, ssem0))):
                j = i * 2 + t
                pltpu.make_async_copy(tab_hbm.at[srcv.at[0]], bw, gsem).wait()
                pltpu.async_copy(bw, acc.at[dstv.at[j]], ssem, add=True)

                @pl.when(j >= 1)
                def _():
                    pltpu.make_async_copy(bo, acc.at[dstv.at[0]],
                                          osem).wait()

                @pl.when(j + 1 < NCHP)
                def _():
                    pltpu.async_copy(tab_hbm.at[srcv.at[j + 1]], bo,
                                     gsem1 if t == 0 else gsem0)
            return 0

        lax.fori_loop(0, NCHP // 2, blk, 0)
        pltpu.make_async_copy(b1, acc.at[dstv.at[0]], ssem1).wait()
    plsc.subcore_barrier()
    pltpu.sync_copy(acc.at[pl.ds(s * RPT, RPT), :],
                    out_hbm.at[c, pl.ds(s * RPT, RPT), :])


def _seg128(src3, dst3, tab):
    mesh = plsc.VectorSubcoreMesh(core_axis_name="c", subcore_axis_name="s")
    return pl.kernel(
        _seg128_body,
        out_type=jax.ShapeDtypeStruct((NC, NPAD, H), jnp.float32),
        mesh=mesh,
        scratch_types=[
            pltpu.VMEM((NCHP, CB), jnp.int32),
            pltpu.VMEM((NCHP, CB), jnp.int32),
            pltpu.VMEM((CB, H), jnp.float32),
            pltpu.VMEM((CB, H), jnp.float32),
            pltpu.VMEM_SHARED((NPAD, H), jnp.float32),
            pltpu.SemaphoreType.DMA,
            pltpu.SemaphoreType.DMA,
            pltpu.SemaphoreType.DMA,
            pltpu.SemaphoreType.DMA,
        ],
    )(src3, dst3, tab)


NR = 8


def _seg1_body(src_hbm, dst_hbm, tab_hbm, out_hbm, srcv, dstv,
               bufs, zb, acc, gsems, ssems, *, gather):
    c = lax.axis_index("c")
    s = lax.axis_index("s")
    wid = c * NS + s
    if gather:
        pltpu.sync_copy(src_hbm.at[wid], srcv)
    pltpu.sync_copy(dst_hbm.at[wid], dstv)

    zero = jnp.zeros((LANES,), jnp.float32)
    one = jnp.ones((LANES,), jnp.float32)

    def vfill(i, _):
        zb[pl.ds(i * LANES, LANES)] = zero
        return 0

    lax.fori_loop(0, RPT // LANES, vfill, 0)
    if not gather:
        for b in bufs:
            for i in range(CB // LANES):
                b[pl.ds(i * LANES, LANES)] = one
    pltpu.sync_copy(zb, acc.at[pl.ds(s * RPT, RPT)])
    plsc.subcore_barrier()

    if gather:
        for t in range(NR // 2):
            pltpu.async_copy(tab_hbm.at[srcv.at[t]], bufs[t], gsems[t])

    def blk(i, _):
        for t in range(NR):
            j = i * NR + t
            bw = bufs[t]
            if gather:
                pltpu.make_async_copy(tab_hbm.at[srcv.at[0]], bw,
                                      gsems[t]).wait()
            else:
                @pl.when(j >= NR)
                def _():
                    pltpu.make_async_copy(bw, acc.at[dstv.at[0]],
                                          ssems[t]).wait()
            pltpu.async_copy(bw, acc.at[dstv.at[j]], ssems[t], add=True)
            if gather:
                t2 = (t + NR // 2) % NR
                bn = bufs[t2]

                @pl.when(j >= NR // 2)
                def _():
                    pltpu.make_async_copy(bn, acc.at[dstv.at[0]],
                                          ssems[t2]).wait()

                @pl.when(j + NR // 2 < NCH)
                def _():
                    pltpu.async_copy(tab_hbm.at[srcv.at[j + NR // 2]], bn,
                                     gsems[t2])
        return 0

    lax.fori_loop(0, NCH // NR, blk, 0)
    drain = range(NR // 2, NR) if gather else range(NR)
    for t in drain:
        pltpu.make_async_copy(bufs[t], acc.at[dstv.at[0]], ssems[t]).wait()
    plsc.subcore_barrier()
    pltpu.sync_copy(acc.at[pl.ds(s * RPT, RPT)], out_hbm.at[c, pl.ds(s * RPT, RPT)])


def _seg1(src3, dst3, tab, gather):
    mesh = plsc.VectorSubcoreMesh(core_axis_name="c", subcore_axis_name="s")
    return pl.kernel(
        functools.partial(_seg1_body, gather=gather),
        out_type=jax.ShapeDtypeStruct((NC, NPAD), jnp.float32),
        mesh=mesh,
        scratch_types=[
            pltpu.VMEM((NCH, CB), jnp.int32),
            pltpu.VMEM((NCH, CB), jnp.int32),
            [pltpu.VMEM((CB,), jnp.float32)] * NR,
            pltpu.VMEM((RPT,), jnp.float32),
            pltpu.VMEM_SHARED((NPAD,), jnp.float32),
            [pltpu.SemaphoreType.DMA] * NR,
            [pltpu.SemaphoreType.DMA] * NR,
        ],
    )(src3, dst3, tab)


def _leaky(x):
    return jnp.where(x >= 0, x, 0.01 * x)


def _pre_body(x_ref, wn_ref, bn_ref, wc_ref, dis_ref, g_ref):
    h = jnp.dot(x_ref[...], wn_ref[...], preferred_element_type=jnp.float32)
    h = h + bn_ref[...]
    t = jnp.dot(h, wc_ref[...], preferred_element_type=jnp.float32)
    g_ref[...] = dis_ref[...] * t


def _layer_body(acc_ref, g_ref, dis_ref, bc_ref, gam_ref, bet_ref, w_ref,
                *rest, last):
    if last:
        w2_ref, h_out, g_out, q_out = rest
    else:
        (g_out,) = rest
    t = dis_ref[...] * (acc_ref[0] + acc_ref[1] + g_ref[...]) + bc_ref[...]
    rid = lax.broadcasted_iota(jnp.int32, (NPAD, 1), 0)
    mask = rid < N
    tm = jnp.where(mask, t, 0.0)
    mu = jnp.sum(tm, axis=0, keepdims=True) / N
    dv = jnp.where(mask, t - mu, 0.0)
    var = jnp.sum(dv * dv, axis=0, keepdims=True) / N
    hn = gam_ref[...] * (t - mu) / jnp.sqrt(var + 1e-5) + bet_ref[...]
    h = _leaky(hn)
    p = jnp.dot(h, w_ref[...], preferred_element_type=jnp.float32)
    if last:
        h_out[...] = h
        g_out[...] = p
        q_out[...] = jnp.dot(h, w2_ref[...], preferred_element_type=jnp.float32)
    else:
        g_out[...] = dis_ref[...] * p


NRW = NPAD // H


def _final_body(scr_ref, h_ref, w1_ref, b1_ref, w2_ref, b2_ref,
                out_ref, selw_scr):
    w = jnp.tanh(scr_ref[...]) + 0.0
    rid = (lax.broadcasted_iota(jnp.int32, (NRW, H), 0) * H
           + lax.broadcasted_iota(jnp.int32, (NRW, H), 1))
    valid = rid < N

    b = lax.bitcast_convert_type(w, jnp.int32)
    key = jnp.where(b >= 0, b, (~b) ^ INT_MIN)
    key = jnp.where(valid, key, INT_MIN)

    def bit_step(i, curm):
        bit = 31 - i
        candm = curm | (jnp.int32(1) << bit)
        candk = candm ^ INT_MIN
        cnt = jnp.sum((key >= candk).astype(jnp.int32))
        return jnp.where(cnt >= K, candm, curm)

    taum = lax.fori_loop(0, 32, bit_step, jnp.int32(0))
    tauk = taum ^ INT_MIN

    gt = key > tauk
    eq = key == tauk
    c_gt = jnp.sum(gt.astype(jnp.int32))
    j_need = K - c_gt

    def cut_step(i, lohi):
        lo, hi = lohi
        mid = (lo + hi) // 2
        cnt = jnp.sum((eq & (rid <= mid)).astype(jnp.int32))
        good = cnt >= j_need
        return (jnp.where(good, lo, mid + 1), jnp.where(good, mid, hi))

    lo, hi = lax.fori_loop(0, 14, cut_step,
                           (jnp.int32(0), jnp.int32(NPAD - 1)))
    sel = gt | (eq & (rid <= hi))
    selw_scr[...] = jnp.where(sel, w, 0.0)

    def mmstep(r, accv):
        row = selw_scr[pl.ds(r, 1), :]
        blk = h_ref[pl.ds(r * H, H), :]
        return accv + jnp.dot(row, blk, preferred_element_type=jnp.float32)

    xg = lax.fori_loop(0, NRW, mmstep, jnp.zeros((1, H), jnp.float32))
    z = jnp.dot(xg, w1_ref[...], preferred_element_type=jnp.float32)
    z = _leaky(z + b1_ref[...])
    z = jnp.dot(z, w2_ref[...], preferred_element_type=jnp.float32)
    out_ref[...] = z + b2_ref[...]


def _tc_call(body, out_shapes, *args, scratch_shapes=()):
    return pl.pallas_call(
        body,
        out_shape=out_shapes,
        scratch_shapes=list(scratch_shapes),
    )(*args)


def kernel(x, edge_attr, edge_index, batch, W_node, b_node, W_edge, b_edge,
           Wc, bc, gamma, beta, Wrel, brel, Wroot, W1, b1, W2, b2):
    src = edge_index[0]
    dst = edge_index[1]
    pad = NW * EPW - E
    padidx = N + (jnp.arange(pad, dtype=jnp.int32) % (NPAD - N))
    src3 = jnp.concatenate([src, padidx]).reshape(NW, NCH, CB)
    dst3 = jnp.concatenate([dst, padidx]).reshape(NW, NCH, CB)

    xpad = jnp.zeros((NPAD, D), jnp.float32).at[:N].set(x)

    deg2 = _seg1(src3, dst3, jnp.zeros((NPAD,), jnp.float32), gather=False)
    dis_col = lax.rsqrt(1.0 + deg2[0] + deg2[1])[:, None]

    g = _tc_call(_pre_body, jax.ShapeDtypeStruct((NPAD, H), jnp.float32),
                 xpad, W_node, b_node.reshape(1, H), Wc[0], dis_col)

    h3 = q = None
    for i in range(3):
        acc = _seg128(src3, dst3, g)
        last = i == 2
        body = functools.partial(_layer_body, last=last)
        args = [acc, g, dis_col, bc[i].reshape(1, H), gamma[i].reshape(1, H),
                beta[i].reshape(1, H)]
        if last:
            out_shapes = [jax.ShapeDtypeStruct((NPAD, H), jnp.float32),
                          jax.ShapeDtypeStruct((NPAD, 1), jnp.float32),
                          jax.ShapeDtypeStruct((NPAD, 1), jnp.float32)]
            h3, g, q = _tc_call(body, out_shapes, *args, Wrel, Wroot)
        else:
            out_shapes = [jax.ShapeDtypeStruct((NPAD, H), jnp.float32)]
            (g,) = _tc_call(body, out_shapes, *args, Wc[i + 1])

    aggw2 = _seg1(src3, dst3, g.reshape(NPAD), gather=True)
    scr = ((aggw2[0] + aggw2[1])[:, None] + q + brel).reshape(NPAD // H, H)

    out = _tc_call(_final_body, jax.ShapeDtypeStruct((1, 1), jnp.float32),
                   scr, h3, W1, b1.reshape(1, H // 2), W2, b2.reshape(1, 1),
                   scratch_shapes=[pltpu.VMEM((NPAD // H, H), jnp.float32)])
    return out


# speedup vs baseline: 1.1290x; 1.1290x over previous
import functools

import jax
import jax.numpy as jnp
from jax import lax
from jax.experimental import pallas as pl
from jax.experimental.pallas import tpu as pltpu
from jax.experimental.pallas import tpu_sc as plsc

N = 10000
E = 320000
D = 128
H = 128
K = 5000
NPAD = 10240
NC = 2
NS = 16
LANES = 16
NW = NC * NS
CB = 128
EPW = 10240
NCH = EPW // CB
RPT = NPAD // NS
ZROWS = 128
INT_MIN = -(2 ** 31)


NPH = 2
NCHP = NCH // NPH


def _seg128_body(src_hbm, dst_hbm, tab_hbm, out_hbm, srcv, dstv,
                 b0, b1, acc, gsem0, gsem1):
    c = lax.axis_index("c")
    s = lax.axis_index("s")
    wid = c * NS + s

    zero = jnp.zeros((LANES,), jnp.float32)

    def zfill(i, _):
        r = i // (H // LANES)
        col = (i % (H // LANES)) * LANES
        b0[r, pl.ds(col, LANES)] = zero
        return 0

    lax.fori_loop(0, ZROWS * (H // LANES), zfill, 0)

    def zcp(i, _):
        pltpu.sync_copy(b0, acc.at[pl.ds(s * RPT + i * ZROWS, ZROWS), :])
        return 0

    lax.fori_loop(0, RPT // ZROWS, zcp, 0)
    plsc.subcore_barrier()

    for ph in range(NPH):
        pltpu.sync_copy(src_hbm.at[wid, pl.ds(ph * NCHP, NCHP)], srcv)
        pltpu.sync_copy(dst_hbm.at[wid, pl.ds(ph * NCHP, NCHP)], dstv)
        pltpu.async_copy(tab_hbm.at[srcv.at[0]], b0, gsem0)
        pltpu.async_copy(tab_hbm.at[srcv.at[1]], b1, gsem1)

        def blk(i, _):
            for t, (bw, sem) in enumerate(((b0, gsem0), (b1, gsem1))):
                j = i * 2 + t
                pltpu.make_async_copy(tab_hbm.at[srcv.at[0]], bw, sem).wait()
                pltpu.sync_copy(bw, acc.at[dstv.at[j]], add=True)

                @pl.when(j + 2 < NCHP)
                def _():
                    pltpu.async_copy(tab_hbm.at[srcv.at[j + 2]], bw, sem)
            return 0

        lax.fori_loop(0, NCHP // 2, blk, 0)
    plsc.subcore_barrier()
    pltpu.sync_copy(acc.at[pl.ds(s * RPT, RPT), :],
                    out_hbm.at[c, pl.ds(s * RPT, RPT), :])


def _seg128(src3, dst3, tab):
    mesh = plsc.VectorSubcoreMesh(core_axis_name="c", subcore_axis_name="s")
    return pl.kernel(
        _seg128_body,
        out_type=jax.ShapeDtypeStruct((NC, NPAD, H), jnp.float32),
        mesh=mesh,
        scratch_types=[
            pltpu.VMEM((NCHP, CB), jnp.int32),
            pltpu.VMEM((NCHP, CB), jnp.int32),
            pltpu.VMEM((CB, H), jnp.float32),
            pltpu.VMEM((CB, H), jnp.float32),
            pltpu.VMEM_SHARED((NPAD, H), jnp.float32),
            pltpu.SemaphoreType.DMA,
            pltpu.SemaphoreType.DMA,
        ],
    )(src3, dst3, tab)


NR = 8


def _seg1_body(src_hbm, dst_hbm, tab_hbm, out_hbm, srcv, dstv,
               bufs, zb, acc, gsems, ssems, *, gather):
    c = lax.axis_index("c")
    s = lax.axis_index("s")
    wid = c * NS + s
    if gather:
        pltpu.sync_copy(src_hbm.at[wid], srcv)
    pltpu.sync_copy(dst_hbm.at[wid], dstv)

    zero = jnp.zeros((LANES,), jnp.float32)
    one = jnp.ones((LANES,), jnp.float32)

    def vfill(i, _):
        zb[pl.ds(i * LANES, LANES)] = zero
        return 0

    lax.fori_loop(0, RPT // LANES, vfill, 0)
    if not gather:
        for b in bufs:
            for i in range(CB // LANES):
                b[pl.ds(i * LANES, LANES)] = one
    pltpu.sync_copy(zb, acc.at[pl.ds(s * RPT, RPT)])
    plsc.subcore_barrier()

    if gather:
        for t in range(NR // 2):
            pltpu.async_copy(tab_hbm.at[srcv.at[t]], bufs[t], gsems[t])

    def blk(i, _):
        for t in range(NR):
            j = i * NR + t
            bw = bufs[t]
            if gather:
                pltpu.make_async_copy(tab_hbm.at[srcv.at[0]], bw,
                                      gsems[t]).wait()
            else:
                @pl.when(j >= NR)
                def _():
                    pltpu.make_async_copy(bw, acc.at[dstv.at[0]],
                                          ssems[t]).wait()
            pltpu.async_copy(bw, acc.at[dstv.at[j]], ssems[t], add=True)
            if gather:
                t2 = (t + NR // 2) % NR
                bn = bufs[t2]

                @pl.when(j >= NR // 2)
                def _():
                    pltpu.make_async_copy(bn, acc.at[dstv.at[0]],
                                          ssems[t2]).wait()

                @pl.when(j + NR // 2 < NCH)
                def _():
                    pltpu.async_copy(tab_hbm.at[srcv.at[j + NR // 2]], bn,
                                     gsems[t2])
        return 0

    lax.fori_loop(0, NCH // NR, blk, 0)
    drain = range(NR // 2, NR) if gather else range(NR)
    for t in drain:
        pltpu.make_async_copy(bufs[t], acc.at[dstv.at[0]], ssems[t]).wait()
    plsc.subcore_barrier()
    pltpu.sync_copy(acc.at[pl.ds(s * RPT, RPT)], out_hbm.at[c, pl.ds(s * RPT, RPT)])


def _seg1(src3, dst3, tab, gather):
    mesh = plsc.VectorSubcoreMesh(core_axis_name="c", subcore_axis_name="s")
    return pl.kernel(
        functools.partial(_seg1_body, gather=gather),
        out_type=jax.ShapeDtypeStruct((NC, NPAD), jnp.float32),
        mesh=mesh,
        scratch_types=[
            pltpu.VMEM((NCH, CB), jnp.int32),
            pltpu.VMEM((NCH, CB), jnp.int32),
            [pltpu.VMEM((CB,), jnp.float32)] * NR,
            pltpu.VMEM((RPT,), jnp.float32),
            pltpu.VMEM_SHARED((NPAD,), jnp.float32),
            [pltpu.SemaphoreType.DMA] * NR,
            [pltpu.SemaphoreType.DMA] * NR,
        ],
    )(src3, dst3, tab)


def _leaky(x):
    return jnp.where(x >= 0, x, 0.01 * x)


def _pre_body(x_ref, wn_ref, bn_ref, wc_ref, dis_ref, g_ref):
    h = jnp.dot(x_ref[...], wn_ref[...], preferred_element_type=jnp.float32)
    h = h + bn_ref[...]
    t = jnp.dot(h, wc_ref[...], preferred_element_type=jnp.float32)
    g_ref[...] = dis_ref[...] * t


def _layer_body(acc_ref, g_ref, dis_ref, bc_ref, gam_ref, bet_ref, w_ref,
                *rest, last):
    if last:
        w2_ref, h_out, g_out, q_out = rest
    else:
        (g_out,) = rest
    t = dis_ref[...] * (acc_ref[0] + acc_ref[1] + g_ref[...]) + bc_ref[...]
    rid = lax.broadcasted_iota(jnp.int32, (NPAD, 1), 0)
    mask = rid < N
    tm = jnp.where(mask, t, 0.0)
    mu = jnp.sum(tm, axis=0, keepdims=True) / N
    dv = jnp.where(mask, t - mu, 0.0)
    var = jnp.sum(dv * dv, axis=0, keepdims=True) / N
    hn = gam_ref[...] * (t - mu) / jnp.sqrt(var + 1e-5) + bet_ref[...]
    h = _leaky(hn)
    p = jnp.dot(h, w_ref[...], preferred_element_type=jnp.float32)
    if last:
        h_out[...] = h
        g_out[...] = p
        q_out[...] = jnp.dot(h, w2_ref[...], preferred_element_type=jnp.float32)
    else:
        g_out[...] = dis_ref[...] * p


NRW = NPAD // H


def _final_body(scr_ref, h_ref, w1_ref, b1_ref, w2_ref, b2_ref,
                out_ref, selw_scr):
    w = jnp.tanh(scr_ref[...]) + 0.0
    rid = (lax.broadcasted_iota(jnp.int32, (NRW, H), 0) * H
           + lax.broadcasted_iota(jnp.int32, (NRW, H), 1))
    valid = rid < N

    b = lax.bitcast_convert_type(w, jnp.int32)
    key = jnp.where(b >= 0, b, (~b) ^ INT_MIN)
    key = jnp.where(valid, key, INT_MIN)

    def bit_step(i, curm):
        bit = 31 - i
        candm = curm | (jnp.int32(1) << bit)
        candk = candm ^ INT_MIN
        cnt = jnp.sum((key >= candk).astype(jnp.int32))
        return jnp.where(cnt >= K, candm, curm)

    taum = lax.fori_loop(0, 32, bit_step, jnp.int32(0))
    tauk = taum ^ INT_MIN

    gt = key > tauk
    eq = key == tauk
    c_gt = jnp.sum(gt.astype(jnp.int32))
    j_need = K - c_gt

    def cut_step(i, lohi):
        lo, hi = lohi
        mid = (lo + hi) // 2
        cnt = jnp.sum((eq & (rid <= mid)).astype(jnp.int32))
        good = cnt >= j_need
        return (jnp.where(good, lo, mid + 1), jnp.where(good, mid, hi))

    lo, hi = lax.fori_loop(0, 14, cut_step,
                           (jnp.int32(0), jnp.int32(NPAD - 1)))
    sel = gt | (eq & (rid <= hi))
    selw_scr[...] = jnp.where(sel, w, 0.0)

    def mmstep(r, accv):
        row = selw_scr[pl.ds(r, 1), :]
        blk = h_ref[pl.ds(r * H, H), :]
        return accv + jnp.dot(row, blk, preferred_element_type=jnp.float32)

    xg = lax.fori_loop(0, NRW, mmstep, jnp.zeros((1, H), jnp.float32))
    z = jnp.dot(xg, w1_ref[...], preferred_element_type=jnp.float32)
    z = _leaky(z + b1_ref[...])
    z = jnp.dot(z, w2_ref[...], preferred_element_type=jnp.float32)
    out_ref[...] = z + b2_ref[...]


def _tc_call(body, out_shapes, *args, scratch_shapes=()):
    return pl.pallas_call(
        body,
        out_shape=out_shapes,
        scratch_shapes=list(scratch_shapes),
    )(*args)


def kernel(x, edge_attr, edge_index, batch, W_node, b_node, W_edge, b_edge,
           Wc, bc, gamma, beta, Wrel, brel, Wroot, W1, b1, W2, b2):
    src = edge_index[0]
    dst = edge_index[1]
    pad = NW * EPW - E
    padidx = N + (jnp.arange(pad, dtype=jnp.int32) % (NPAD - N))
    src3 = jnp.concatenate([src, padidx]).reshape(NW, NCH, CB)
    dst3 = jnp.concatenate([dst, padidx]).reshape(NW, NCH, CB)

    xpad = jnp.zeros((NPAD, D), jnp.float32).at[:N].set(x)

    deg2 = _seg1(src3, dst3, jnp.zeros((NPAD,), jnp.float32), gather=False)
    dis_col = lax.rsqrt(1.0 + deg2[0] + deg2[1])[:, None]

    g = _tc_call(_pre_body, jax.ShapeDtypeStruct((NPAD, H), jnp.float32),
                 xpad, W_node, b_node.reshape(1, H), Wc[0], dis_col)

    h3 = q = None
    for i in range(3):
        acc = _seg128(src3, dst3, g)
        last = i == 2
        body = functools.partial(_layer_body, last=last)
        args = [acc, g, dis_col, bc[i].reshape(1, H), gamma[i].reshape(1, H),
                beta[i].reshape(1, H)]
        if last:
            out_shapes = [jax.ShapeDtypeStruct((NPAD, H), jnp.float32),
                          jax.ShapeDtypeStruct((NPAD, 1), jnp.float32),
                          jax.ShapeDtypeStruct((NPAD, 1), jnp.float32)]
            h3, g, q = _tc_call(body, out_shapes, *args, Wrel, Wroot)
        else:
            out_shapes = [jax.ShapeDtypeStruct((NPAD, H), jnp.float32)]
            (g,) = _tc_call(body, out_shapes, *args, Wc[i + 1])

    aggw2 = _seg1(src3, dst3, g.reshape(NPAD), gather=True)
    scr = ((aggw2[0] + aggw2[1])[:, None] + q + brel).reshape(NPAD // H, H)

    out = _tc_call(_final_body, jax.ShapeDtypeStruct((1, 1), jnp.float32),
                   scr, h3, W1, b1.reshape(1, H // 2), W2, b2.reshape(1, 1),
                   scratch_shapes=[pltpu.VMEM((NPAD // H, H), jnp.float32)])
    return out


# speedup vs baseline: 1.1501x; 1.0187x over previous
import functools

import jax
import jax.numpy as jnp
from jax import lax
from jax.experimental import pallas as pl
from jax.experimental.pallas import tpu as pltpu
from jax.experimental.pallas import tpu_sc as plsc

N = 10000
E = 320000
D = 128
H = 128
K = 5000
NPAD = 10240
NC = 2
NS = 16
LANES = 16
NW = NC * NS
CB = 128
EPW = 10240
NCH = EPW // CB
RPT = NPAD // NS
ZROWS = 128
INT_MIN = -(2 ** 31)


NPH = 2
NCHP = NCH // NPH


def _seg128_body(ei_hbm, tab_hbm, out_hbm, srcv, dstv,
                 b0, b1, acc, gsem0, gsem1):
    c = lax.axis_index("c")
    s = lax.axis_index("s")
    wid = c * NS + s

    zero = jnp.zeros((LANES,), jnp.float32)

    def zfill(i, _):
        r = i // (H // LANES)
        col = (i % (H // LANES)) * LANES
        b0[r, pl.ds(col, LANES)] = zero
        return 0

    lax.fori_loop(0, ZROWS * (H // LANES), zfill, 0)

    def zcp(i, _):
        pltpu.sync_copy(b0, acc.at[pl.ds(s * RPT + i * ZROWS, ZROWS), :])
        return 0

    lax.fori_loop(0, RPT // ZROWS, zcp, 0)
    plsc.subcore_barrier()

    for ph in range(NPH):
        pltpu.sync_copy(ei_hbm.at[0, wid, pl.ds(ph * NCHP, NCHP)], srcv)
        pltpu.sync_copy(ei_hbm.at[1, wid, pl.ds(ph * NCHP, NCHP)], dstv)
        pltpu.async_copy(tab_hbm.at[srcv.at[0]], b0, gsem0)
        pltpu.async_copy(tab_hbm.at[srcv.at[1]], b1, gsem1)

        def blk(i, _):
            for t, (bw, sem) in enumerate(((b0, gsem0), (b1, gsem1))):
                j = i * 2 + t
                pltpu.make_async_copy(tab_hbm.at[srcv.at[0]], bw, sem).wait()
                pltpu.sync_copy(bw, acc.at[dstv.at[j]], add=True)

                @pl.when(j + 2 < NCHP)
                def _():
                    pltpu.async_copy(tab_hbm.at[srcv.at[j + 2]], bw, sem)
            return 0

        lax.fori_loop(0, NCHP // 2, blk, 0)
    plsc.subcore_barrier()
    pltpu.sync_copy(acc.at[pl.ds(s * RPT, RPT), :],
                    out_hbm.at[c, pl.ds(s * RPT, RPT), :])


def _seg128(ei4, tab):
    mesh = plsc.VectorSubcoreMesh(core_axis_name="c", subcore_axis_name="s")
    return pl.kernel(
        _seg128_body,
        out_type=jax.ShapeDtypeStruct((NC, NPAD, H), jnp.float32),
        mesh=mesh,
        scratch_types=[
            pltpu.VMEM((NCHP, CB), jnp.int32),
            pltpu.VMEM((NCHP, CB), jnp.int32),
            pltpu.VMEM((CB, H), jnp.float32),
            pltpu.VMEM((CB, H), jnp.float32),
            pltpu.VMEM_SHARED((NPAD, H), jnp.float32),
            pltpu.SemaphoreType.DMA,
            pltpu.SemaphoreType.DMA,
        ],
    )(ei4, tab)


NR = 8


def _seg1_body(ei_hbm, tab_hbm, out_hbm, srcv, dstv,
               bufs, zb, acc, gsems, ssems, *, gather):
    c = lax.axis_index("c")
    s = lax.axis_index("s")
    wid = c * NS + s
    if gather:
        pltpu.sync_copy(ei_hbm.at[0, wid], srcv)
    pltpu.sync_copy(ei_hbm.at[1, wid], dstv)

    zero = jnp.zeros((LANES,), jnp.float32)
    one = jnp.ones((LANES,), jnp.float32)

    def vfill(i, _):
        zb[pl.ds(i * LANES, LANES)] = zero
        return 0

    lax.fori_loop(0, RPT // LANES, vfill, 0)
    if not gather:
        for b in bufs:
            for i in range(CB // LANES):
                b[pl.ds(i * LANES, LANES)] = one
    pltpu.sync_copy(zb, acc.at[pl.ds(s * RPT, RPT)])
    plsc.subcore_barrier()

    if gather:
        for t in range(NR // 2):
            pltpu.async_copy(tab_hbm.at[srcv.at[t]], bufs[t], gsems[t])

    def blk(i, _):
        for t in range(NR):
            j = i * NR + t
            bw = bufs[t]
            if gather:
                pltpu.make_async_copy(tab_hbm.at[srcv.at[0]], bw,
                                      gsems[t]).wait()
            else:
                @pl.when(j >= NR)
                def _():
                    pltpu.make_async_copy(bw, acc.at[dstv.at[0]],
                                          ssems[t]).wait()
            pltpu.async_copy(bw, acc.at[dstv.at[j]], ssems[t], add=True)
            if gather:
                t2 = (t + NR // 2) % NR
                bn = bufs[t2]

                @pl.when(j >= NR // 2)
                def _():
                    pltpu.make_async_copy(bn, acc.at[dstv.at[0]],
                                          ssems[t2]).wait()

                @pl.when(j + NR // 2 < NCH)
                def _():
                    pltpu.async_copy(tab_hbm.at[srcv.at[j + NR // 2]], bn,
                                     gsems[t2])
        return 0

    lax.fori_loop(0, NCH // NR, blk, 0)
    drain = range(NR // 2, NR) if gather else range(NR)
    for t in drain:
        pltpu.make_async_copy(bufs[t], acc.at[dstv.at[0]], ssems[t]).wait()
    plsc.subcore_barrier()
    pltpu.sync_copy(acc.at[pl.ds(s * RPT, RPT)], out_hbm.at[c, pl.ds(s * RPT, RPT)])


def _seg1(ei4, tab, gather):
    mesh = plsc.VectorSubcoreMesh(core_axis_name="c", subcore_axis_name="s")
    return pl.kernel(
        functools.partial(_seg1_body, gather=gather),
        out_type=jax.ShapeDtypeStruct((NC, NPAD), jnp.float32),
        mesh=mesh,
        scratch_types=[
            pltpu.VMEM((NCH, CB), jnp.int32),
            pltpu.VMEM((NCH, CB), jnp.int32),
            [pltpu.VMEM((CB,), jnp.float32)] * NR,
            pltpu.VMEM((RPT,), jnp.float32),
            pltpu.VMEM_SHARED((NPAD,), jnp.float32),
            [pltpu.SemaphoreType.DMA] * NR,
            [pltpu.SemaphoreType.DMA] * NR,
        ],
    )(ei4, tab)


def _leaky(x):
    return jnp.where(x >= 0, x, 0.01 * x)


def _pre_body(x_ref, wn_ref, bn_ref, wc_ref, dis_ref, g_ref):
    h = jnp.dot(x_ref[...], wn_ref[...], preferred_element_type=jnp.float32)
    h = h + bn_ref[...]
    t = jnp.dot(h, wc_ref[...], preferred_element_type=jnp.float32)
    g_ref[pl.ds(0, N), :] = dis_ref[pl.ds(0, N), :] * t
    g_ref[pl.ds(N, NPAD - N), :] = jnp.zeros((NPAD - N, H), jnp.float32)


def _layer_body(acc_ref, g_ref, dis_ref, bc_ref, gam_ref, bet_ref, w_ref,
                *rest, last):
    if last:
        w2_ref, h_out, g_out, q_out = rest
    else:
        (g_out,) = rest
    t = dis_ref[...] * (acc_ref[0] + acc_ref[1] + g_ref[...]) + bc_ref[...]
    rid = lax.broadcasted_iota(jnp.int32, (NPAD, 1), 0)
    mask = rid < N
    tm = jnp.where(mask, t, 0.0)
    mu = jnp.sum(tm, axis=0, keepdims=True) / N
    dv = jnp.where(mask, t - mu, 0.0)
    var = jnp.sum(dv * dv, axis=0, keepdims=True) / N
    hn = gam_ref[...] * (t - mu) / jnp.sqrt(var + 1e-5) + bet_ref[...]
    h = _leaky(hn)
    p = jnp.dot(h, w_ref[...], preferred_element_type=jnp.float32)
    if last:
        h_out[...] = h
        g_out[...] = p
        q_out[...] = jnp.dot(h, w2_ref[...], preferred_element_type=jnp.float32)
    else:
        g_out[...] = dis_ref[...] * p


NRW = NPAD // H


def _final_body(scr_ref, h_ref, w1_ref, b1_ref, w2_ref, b2_ref,
                out_ref, selw_scr):
    w = jnp.tanh(scr_ref[...]) + 0.0
    rid = (lax.broadcasted_iota(jnp.int32, (NRW, H), 0) * H
           + lax.broadcasted_iota(jnp.int32, (NRW, H), 1))
    valid = rid < N

    b = lax.bitcast_convert_type(w, jnp.int32)
    key = jnp.where(b >= 0, b, (~b) ^ INT_MIN)
    key = jnp.where(valid, key, INT_MIN)

    def bit_step(i, curm):
        bit = 31 - i
        candm = curm | (jnp.int32(1) << bit)
        candk = candm ^ INT_MIN
        cnt = jnp.sum((key >= candk).astype(jnp.int32))
        return jnp.where(cnt >= K, candm, curm)

    taum = lax.fori_loop(0, 32, bit_step, jnp.int32(0))
    tauk = taum ^ INT_MIN

    gt = key > tauk
    eq = key == tauk
    c_gt = jnp.sum(gt.astype(jnp.int32))
    j_need = K - c_gt

    def cut_step(i, lohi):
        lo, hi = lohi
        mid = (lo + hi) // 2
        cnt = jnp.sum((eq & (rid <= mid)).astype(jnp.int32))
        good = cnt >= j_need
        return (jnp.where(good, lo, mid + 1), jnp.where(good, mid, hi))

    lo, hi = lax.fori_loop(0, 14, cut_step,
                           (jnp.int32(0), jnp.int32(NPAD - 1)))
    sel = gt | (eq & (rid <= hi))
    selw_scr[...] = jnp.where(sel, w, 0.0)

    def mmstep(r, accv):
        row = selw_scr[pl.ds(r, 1), :]
        blk = h_ref[pl.ds(r * H, H), :]
        return accv + jnp.dot(row, blk, preferred_element_type=jnp.float32)

    xg = lax.fori_loop(0, NRW, mmstep, jnp.zeros((1, H), jnp.float32))
    z = jnp.dot(xg, w1_ref[...], preferred_element_type=jnp.float32)
    z = _leaky(z + b1_ref[...])
    z = jnp.dot(z, w2_ref[...], preferred_element_type=jnp.float32)
    out_ref[...] = z + b2_ref[...]


def _tc_call(body, out_shapes, *args, scratch_shapes=()):
    return pl.pallas_call(
        body,
        out_shape=out_shapes,
        scratch_shapes=list(scratch_shapes),
    )(*args)


def kernel(x, edge_attr, edge_index, batch, W_node, b_node, W_edge, b_edge,
           Wc, bc, gamma, beta, Wrel, brel, Wroot, W1, b1, W2, b2):
    pad = NW * EPW - E
    padidx = N + (jnp.arange(pad, dtype=jnp.int32) % (NPAD - N))
    ei4 = jnp.concatenate(
        [edge_index, jnp.broadcast_to(padidx, (2, pad))], axis=1,
    ).reshape(2, NW, NCH, CB)

    deg2 = _seg1(ei4, jnp.zeros((NPAD,), jnp.float32), gather=False)
    dis_col = lax.rsqrt(1.0 + deg2[0] + deg2[1])[:, None]

    g = _tc_call(_pre_body, jax.ShapeDtypeStruct((NPAD, H), jnp.float32),
                 x, W_node, b_node.reshape(1, H), Wc[0], dis_col)

    h3 = q = None
    for i in range(3):
        acc = _seg128(ei4, g)
        last = i == 2
        body = functools.partial(_layer_body, last=last)
        args = [acc, g, dis_col, bc[i].reshape(1, H), gamma[i].reshape(1, H),
                beta[i].reshape(1, H)]
        if last:
            out_shapes = [jax.ShapeDtypeStruct((NPAD, H), jnp.float32),
                          jax.ShapeDtypeStruct((NPAD, 1), jnp.float32),
                          jax.ShapeDtypeStruct((NPAD, 1), jnp.float32)]
            h3, g, q = _tc_call(body, out_shapes, *args, Wrel, Wroot)
        else:
            out_shapes = [jax.ShapeDtypeStruct((NPAD, H), jnp.float32)]
            (g,) = _tc_call(body, out_shapes, *args, Wc[i + 1])

    aggw2 = _seg1(ei4, g.reshape(NPAD), gather=True)
    scr = ((aggw2[0] + aggw2[1])[:, None] + q + brel).reshape(NPAD // H, H)

    out = _tc_call(_final_body, jax.ShapeDtypeStruct((1, 1), jnp.float32),
                   scr, h3, W1, b1.reshape(1, H // 2), W2, b2.reshape(1, 1),
                   scratch_shapes=[pltpu.VMEM((NPAD // H, H), jnp.float32)])
    return out
